# Initial kernel scaffold; baseline (speedup 1.0000x reference)
import jax
import jax.numpy as jnp
from jax.experimental import pallas as pl


def kernel(x, edge_index, batch, W1, b1, gamma1, beta1, W2, b2, eps, W_out, b_out):
    raise NotImplementedError("write your pallas kernel here")


# speedup vs baseline: 4.5711x; 4.5711x over previous
import functools

import jax
import jax.numpy as jnp
from jax import lax
from jax.experimental import pallas as pl
from jax.experimental.pallas import tpu as pltpu
from jax.experimental.pallas import tpu_sc as plsc

N = 10000
E = 320000
D = 128
G = 128
C = 10
NUM_LAYERS = 3

_NC = 2
_NS = 16
_NW = _NC * _NS
_EPW = E // _NW
_K = 80
_STEPS = _EPW // _K
_RPS = 624
_REM = N - _NS * _RPS


def _mp_body(x_hbm, src_hbm, dst_hbm, out_hbm, src_v, dst_v, rows_v, agg_sh, sem):
    c = lax.axis_index("c")
    s = lax.axis_index("s")
    wid = s * _NC + c
    pltpu.sync_copy(x_hbm.at[pl.ds(s * _RPS, _RPS)],
                    agg_sh.at[pl.ds(s * _RPS, _RPS)])

    @pl.when(s == 0)
    def _():
        pltpu.sync_copy(x_hbm.at[pl.ds(_NS * _RPS, _REM)],
                        agg_sh.at[pl.ds(_NS * _RPS, _REM)])

    plsc.subcore_barrier()

    base0 = wid * _EPW

    def step(g, carry):
        base = base0 + g * _K
        pltpu.sync_copy(src_hbm.at[pl.ds(base, _K)], src_v)
        pltpu.sync_copy(dst_hbm.at[pl.ds(base, _K)], dst_v)
        pltpu.async_copy(x_hbm.at[src_v], rows_v, sem).wait()
        pltpu.sync_copy(rows_v, agg_sh.at[dst_v], add=True)
        return carry

    lax.fori_loop(0, _STEPS, step, 0)
    plsc.subcore_barrier()
    pltpu.sync_copy(agg_sh.at[pl.ds(s * _RPS, _RPS)],
                    out_hbm.at[pl.ds(c * N + s * _RPS, _RPS)])

    @pl.when(s == 0)
    def _():
        pltpu.sync_copy(agg_sh.at[pl.ds(_NS * _RPS, _REM)],
                        out_hbm.at[pl.ds(c * N + _NS * _RPS, _REM)])


_mp_call = functools.partial(
    pl.kernel,
    out_type=jax.ShapeDtypeStruct((2 * N, D), jnp.float32),
    mesh=plsc.VectorSubcoreMesh(core_axis_name="c", subcore_axis_name="s"),
    scratch_types=[
        pltpu.VMEM((_K,), jnp.int32),
        pltpu.VMEM((_K,), jnp.int32),
        pltpu.VMEM((_K, D), jnp.float32),
        pltpu.VMEM_SHARED((N, D), jnp.float32),
        pltpu.SemaphoreType.DMA,
    ],
)(_mp_body)


def _dense_body(last, x_ref, p0_ref, p1_ref, w1_ref, b1_ref, g_ref, bt_ref,
                w2_ref, b2_ref, em1_ref, *rest):
    if last:
        batch_ref, wout_ref, bout_ref, o_ref = rest
    else:
        (o_ref,) = rest
    h = p0_ref[...] + p1_ref[...] + em1_ref[0, 0] * x_ref[...]
    h = jnp.dot(h, w1_ref[...], preferred_element_type=jnp.float32) + b1_ref[...]
    mu = jnp.mean(h, axis=0, keepdims=True)
    var = jnp.mean((h - mu) ** 2, axis=0, keepdims=True)
    h = (h - mu) * lax.rsqrt(var + 1e-5) * g_ref[...] + bt_ref[...]
    h = jnp.maximum(h, 0.0)
    h = jnp.dot(h, w2_ref[...], preferred_element_type=jnp.float32) + b2_ref[...]
    if not last:
        o_ref[...] = jnp.maximum(h, 0.0)
        return
    gids = lax.broadcasted_iota(jnp.int32, (1, G), 1)
    onehot = (batch_ref[...] == gids).astype(jnp.float32)
    sums = lax.dot_general(onehot, h, (((0,), (0,)), ((), ())),
                           preferred_element_type=jnp.float32)
    cnt = lax.dot_general(onehot, jnp.ones((N, 1), jnp.float32),
                          (((0,), (0,)), ((), ())),
                          preferred_element_type=jnp.float32)
    hg = sums / jnp.maximum(cnt, 1.0)
    o_ref[...] = (jnp.dot(hg, wout_ref[...], preferred_element_type=jnp.float32)
                  + bout_ref[...])


_dense_mid = pl.pallas_call(
    functools.partial(_dense_body, False),
    out_shape=jax.ShapeDtypeStruct((N, D), jnp.float32),
)

_dense_last = pl.pallas_call(
    functools.partial(_dense_body, True),
    out_shape=jax.ShapeDtypeStruct((G, C), jnp.float32),
)


def kernel(x, edge_index, batch, W1, b1, gamma1, beta1, W2, b2, eps, W_out, b_out):
    src = edge_index[0].astype(jnp.int32)
    dst = edge_index[1].astype(jnp.int32)
    batch2 = batch.astype(jnp.int32).reshape(N, 1)
    cur = x
    for l in range(NUM_LAYERS):
        parts = _mp_call(cur, src, dst)
        p0, p1 = parts[:N], parts[N:]
        em1 = (eps[l] - 1.0).reshape(1, 1)
        args = (cur, p0, p1, W1[l], b1[l].reshape(1, D), gamma1[l].reshape(1, D),
                beta1[l].reshape(1, D), W2[l], b2[l].reshape(1, D), em1)
        if l < NUM_LAYERS - 1:
            cur = _dense_mid(*args)
        else:
            out = _dense_last(*args, batch2, W_out, b_out.reshape(1, C))
    return out
